# baseline (speedup 1.0000x reference)
import functools

import jax
import jax.numpy as jnp
from jax import lax
from jax.experimental import pallas as pl
from jax.experimental.pallas import tpu as pltpu
from jax.experimental.pallas import tpu_sc as plsc

N = 100000
B = 16384
D = 64
UF = 13
EF = 13
NC = 2
NS = 16
NW = NC * NS
CHUNK = 128
CPW = B // (NW * CHUNK)
BPW = B // NW

_mesh = plsc.VectorSubcoreMesh(core_axis_name="c", subcore_axis_name="s")


@functools.partial(
    pl.kernel,
    mesh=_mesh,
    out_type=jax.ShapeDtypeStruct((B, 2 * D), jnp.float32),
    scratch_types=[
        pltpu.VMEM((CPW, CHUNK), jnp.int32),
        pltpu.VMEM((BPW, 2 * D), jnp.float32),
        pltpu.SemaphoreType.DMA,
    ],
    compiler_params=pltpu.CompilerParams(use_tc_tiling_on_sc=True),
)
def _sc_gather(idx_hbm, cat_hbm, out_hbm, idx_v, rows_v, sem):
    wid = lax.axis_index("s") * NC + lax.axis_index("c")
    base = wid * BPW
    pltpu.sync_copy(idx_hbm.at[pl.ds(wid * CPW, CPW)], idx_v)
    copies = []
    for j in range(CPW):
        copies.append(
            pltpu.async_copy(
                cat_hbm.at[idx_v.at[j]], rows_v.at[pl.ds(j * CHUNK, CHUNK)], sem
            )
        )
    for cp in copies:
        cp.wait()
    pltpu.sync_copy(rows_v, out_hbm.at[pl.ds(base, BPW)])


TBLK = 1024


def _t1_body(persT, commT, cat):
    cat[...] = jnp.concatenate(
        [jnp.swapaxes(persT[...], 0, 1), jnp.swapaxes(commT[...], 0, 1)], axis=1
    )


_t1 = pl.pallas_call(
    _t1_body,
    grid=(pl.cdiv(N, TBLK),),
    in_specs=[
        pl.BlockSpec((D, TBLK), lambda i: (0, i)),
        pl.BlockSpec((D, TBLK), lambda i: (0, i)),
    ],
    out_specs=pl.BlockSpec((TBLK, 2 * D), lambda i: (i, 0)),
    out_shape=jax.ShapeDtypeStruct((N, 2 * D), jnp.float32),
)


BLK = 2048


def _full(shape):
    return pl.BlockSpec(shape, lambda i: (0, 0))


def _t2a_body(ufT, efT, uw1, ub1, uw2, ub2, ew1, eb1, ew2, eb2, aw, ab, s_out):
    f32 = jnp.float32
    dn = (((0,), (0,)), ((), ()))
    h = jnp.maximum(
        lax.dot_general(uw1[...], ufT[...], dn, preferred_element_type=f32)
        + ub1[...], 0.0)
    ue = jnp.maximum(
        lax.dot_general(uw2[...], h, dn, preferred_element_type=f32)
        + ub2[...], 0.0)
    g = jnp.maximum(
        lax.dot_general(ew1[...], efT[...], dn, preferred_element_type=f32)
        + eb1[...], 0.0)
    ee = jnp.maximum(
        lax.dot_general(ew2[...], g, dn, preferred_element_type=f32)
        + eb2[...], 0.0)
    z = 0.5 * ue + 0.3 * ee
    s_out[...] = (
        lax.dot_general(aw[...], z, dn, preferred_element_type=f32) + ab[...]
    )


_t2a = pl.pallas_call(
    _t2a_body,
    grid=(B // BLK,),
    in_specs=[
        pl.BlockSpec((UF, BLK), lambda i: (0, i)),
        pl.BlockSpec((EF, BLK), lambda i: (0, i)),
        _full((UF, 64)),
        _full((64, 1)),
        _full((64, D)),
        _full((D, 1)),
        _full((EF, 64)),
        _full((64, 1)),
        _full((64, D)),
        _full((D, 1)),
        _full((D, 1)),
        _full((1, 1)),
    ],
    out_specs=pl.BlockSpec((1, BLK), lambda i: (0, i)),
    out_shape=jax.ShapeDtypeStruct((1, B), jnp.float32),
)


def _t2b_body(both, s, aw, ratingT, ipT, icT):
    f32 = jnp.float32
    bothT = jnp.swapaxes(both[...], 0, 1)
    ipt = bothT[:D]
    ict = bothT[D:]
    ipT[...] = ipt
    icT[...] = ict
    dn = (((0,), (0,)), ((), ()))
    logits = lax.dot_general(
        aw[...], ipt + ict, dn, preferred_element_type=f32
    ) + s[...]
    ratingT[...] = jax.nn.sigmoid(logits)


_t2b = pl.pallas_call(
    _t2b_body,
    grid=(B // BLK,),
    in_specs=[
        pl.BlockSpec((BLK, 2 * D), lambda i: (i, 0)),
        pl.BlockSpec((1, BLK), lambda i: (0, i)),
        _full((D, 1)),
    ],
    out_specs=[
        pl.BlockSpec((1, BLK), lambda i: (0, i)),
        pl.BlockSpec((D, BLK), lambda i: (0, i)),
        pl.BlockSpec((D, BLK), lambda i: (0, i)),
    ],
    out_shape=[
        jax.ShapeDtypeStruct((1, B), jnp.float32),
        jax.ShapeDtypeStruct((D, B), jnp.float32),
        jax.ShapeDtypeStruct((D, B), jnp.float32),
    ],
)


def kernel(item_indices, user_features, exercise_features,
           item_personality, item_commonality,
           u_w1, u_b1, u_w2, u_b2,
           e_w1, e_b1, e_w2, e_b2,
           a_w, a_b):
    idx2 = item_indices.astype(jnp.int32).reshape(B // CHUNK, CHUNK)
    cat = _t1(item_personality.T, item_commonality.T)
    both = _sc_gather(idx2, cat)
    sT = _t2a(
        user_features.T, exercise_features.T,
        u_w1, u_b1.reshape(64, 1), u_w2, u_b2.reshape(D, 1),
        e_w1, e_b1.reshape(64, 1), e_w2, e_b2.reshape(D, 1),
        a_w, a_b.reshape(1, 1),
    )
    ratingT, ipT, icT = _t2b(both, sT, a_w)
    return (ratingT.T, ipT.T, icT.T)


# speedup vs baseline: 1.2832x; 1.2832x over previous
import functools

import jax
import jax.numpy as jnp
from jax import lax
from jax.experimental import pallas as pl
from jax.experimental.pallas import tpu as pltpu
from jax.experimental.pallas import tpu_sc as plsc

N = 100000
B = 16384
D = 64
UF = 13
EF = 13
NC = 2
NS = 16
NW = NC * NS
CHUNK = 128
CPW = B // (NW * CHUNK)
BPW = B // NW

_mesh = plsc.VectorSubcoreMesh(core_axis_name="c", subcore_axis_name="s")


@functools.partial(
    pl.kernel,
    mesh=_mesh,
    out_type=jax.ShapeDtypeStruct((B, 2 * D), jnp.float32),
    scratch_types=[
        pltpu.VMEM((CPW, CHUNK), jnp.int32),
        pltpu.VMEM((BPW, 2 * D), jnp.float32),
        pltpu.SemaphoreType.DMA,
    ],
    compiler_params=pltpu.CompilerParams(use_tc_tiling_on_sc=True),
)
def _sc_gather(idx_hbm, cat_hbm, out_hbm, idx_v, rows_v, sem):
    wid = lax.axis_index("s") * NC + lax.axis_index("c")
    base = wid * BPW
    pltpu.sync_copy(idx_hbm.at[pl.ds(wid * CPW, CPW)], idx_v)
    copies = []
    for j in range(CPW):
        copies.append(
            pltpu.async_copy(
                cat_hbm.at[idx_v.at[j]], rows_v.at[pl.ds(j * CHUNK, CHUNK)], sem
            )
        )
    for cp in copies:
        cp.wait()
    pltpu.sync_copy(rows_v, out_hbm.at[pl.ds(base, BPW)])


TBLK = 2048
_DN0 = (((0,), (0,)), ((), ()))


def _t1_body(persT, commT, cat):
    f32 = jnp.float32
    eye = jnp.eye(D, dtype=f32)
    cat[:, :D] = lax.dot_general(persT[...], eye, _DN0, preferred_element_type=f32)
    cat[:, D:] = lax.dot_general(commT[...], eye, _DN0, preferred_element_type=f32)


_t1 = pl.pallas_call(
    _t1_body,
    grid=(pl.cdiv(N, TBLK),),
    in_specs=[
        pl.BlockSpec((D, TBLK), lambda i: (0, i)),
        pl.BlockSpec((D, TBLK), lambda i: (0, i)),
    ],
    out_specs=pl.BlockSpec((TBLK, 2 * D), lambda i: (i, 0)),
    out_shape=jax.ShapeDtypeStruct((N, 2 * D), jnp.float32),
)


BLK = 2048


def _full(shape):
    return pl.BlockSpec(shape, lambda i: (0, 0))


def _t2_body(ufT, efT, both, uw1, ub1, uw2, ub2, ew1, eb1, ew2, eb2, aw, ab,
             ratingT, ipT, icT):
    f32 = jnp.float32
    h = jnp.maximum(
        lax.dot_general(uw1[...], ufT[...], _DN0, preferred_element_type=f32)
        + ub1[...], 0.0)
    ue = jnp.maximum(
        lax.dot_general(uw2[...], h, _DN0, preferred_element_type=f32)
        + ub2[...], 0.0)
    g = jnp.maximum(
        lax.dot_general(ew1[...], efT[...], _DN0, preferred_element_type=f32)
        + eb1[...], 0.0)
    ee = jnp.maximum(
        lax.dot_general(ew2[...], g, _DN0, preferred_element_type=f32)
        + eb2[...], 0.0)
    z = 0.5 * ue + 0.3 * ee
    bothT = jnp.swapaxes(both[...], 0, 1)
    ipt = bothT[:D]
    ict = bothT[D:]
    ipT[...] = ipt
    icT[...] = ict
    logits = lax.dot_general(
        aw[...], ipt + ict + z, _DN0, preferred_element_type=f32
    ) + ab[...]
    ratingT[...] = jax.nn.sigmoid(logits)


_t2 = pl.pallas_call(
    _t2_body,
    grid=(B // BLK,),
    in_specs=[
        pl.BlockSpec((UF, BLK), lambda i: (0, i)),
        pl.BlockSpec((EF, BLK), lambda i: (0, i)),
        pl.BlockSpec((BLK, 2 * D), lambda i: (i, 0)),
        _full((UF, 64)),
        _full((64, 1)),
        _full((64, D)),
        _full((D, 1)),
        _full((EF, 64)),
        _full((64, 1)),
        _full((64, D)),
        _full((D, 1)),
        _full((D, 1)),
        _full((1, 1)),
    ],
    out_specs=[
        pl.BlockSpec((1, BLK), lambda i: (0, i)),
        pl.BlockSpec((D, BLK), lambda i: (0, i)),
        pl.BlockSpec((D, BLK), lambda i: (0, i)),
    ],
    out_shape=[
        jax.ShapeDtypeStruct((1, B), jnp.float32),
        jax.ShapeDtypeStruct((D, B), jnp.float32),
        jax.ShapeDtypeStruct((D, B), jnp.float32),
    ],
)


def kernel(item_indices, user_features, exercise_features,
           item_personality, item_commonality,
           u_w1, u_b1, u_w2, u_b2,
           e_w1, e_b1, e_w2, e_b2,
           a_w, a_b):
    idx2 = item_indices.astype(jnp.int32).reshape(B // CHUNK, CHUNK)
    cat = _t1(item_personality.T, item_commonality.T)
    both = _sc_gather(idx2, cat)
    ratingT, ipT, icT = _t2(
        user_features.T, exercise_features.T, both,
        u_w1, u_b1.reshape(64, 1), u_w2, u_b2.reshape(D, 1),
        e_w1, e_b1.reshape(64, 1), e_w2, e_b2.reshape(D, 1),
        a_w, a_b.reshape(1, 1),
    )
    return (ratingT.T, ipT.T, icT.T)


# speedup vs baseline: 1.4995x; 1.1686x over previous
import functools

import jax
import jax.numpy as jnp
from jax import lax
from jax.experimental import pallas as pl
from jax.experimental.pallas import tpu as pltpu
from jax.experimental.pallas import tpu_sc as plsc

N = 100000
B = 16384
D = 64
UF = 13
EF = 13
NC = 2
NS = 16
NW = NC * NS
CHUNK = 128
CPW = B // (NW * CHUNK)
BPW = B // NW

_mesh = plsc.VectorSubcoreMesh(core_axis_name="c", subcore_axis_name="s")


@functools.partial(
    pl.kernel,
    mesh=_mesh,
    out_type=jax.ShapeDtypeStruct((B, 2 * D), jnp.float32),
    scratch_types=[
        pltpu.VMEM((CPW, CHUNK), jnp.int32),
        pltpu.VMEM((BPW, 2 * D), jnp.float32),
        pltpu.SemaphoreType.DMA,
    ],
    compiler_params=pltpu.CompilerParams(use_tc_tiling_on_sc=True),
)
def _sc_gather(idx_hbm, cat_hbm, out_hbm, idx_v, rows_v, sem):
    wid = lax.axis_index("s") * NC + lax.axis_index("c")
    base = wid * BPW
    pltpu.sync_copy(idx_hbm.at[pl.ds(wid * CPW, CPW)], idx_v)
    copies = []
    for j in range(CPW):
        copies.append(
            pltpu.async_copy(
                cat_hbm.at[idx_v.at[j]], rows_v.at[pl.ds(j * CHUNK, CHUNK)], sem
            )
        )
    for cp in copies:
        cp.wait()
    pltpu.sync_copy(rows_v, out_hbm.at[pl.ds(base, BPW)])


TBLK = 4096
_DN0 = (((0,), (0,)), ((), ()))


def _t1_body(persT, commT, cat):
    f32 = jnp.float32
    eye = jnp.eye(D, dtype=f32)
    cat[:, :D] = lax.dot_general(persT[...], eye, _DN0, preferred_element_type=f32)
    cat[:, D:] = lax.dot_general(commT[...], eye, _DN0, preferred_element_type=f32)


_t1 = pl.pallas_call(
    _t1_body,
    grid=(pl.cdiv(N, TBLK),),
    in_specs=[
        pl.BlockSpec((D, TBLK), lambda i: (0, i)),
        pl.BlockSpec((D, TBLK), lambda i: (0, i)),
    ],
    out_specs=pl.BlockSpec((TBLK, 2 * D), lambda i: (i, 0)),
    out_shape=jax.ShapeDtypeStruct((N, 2 * D), jnp.float32),
)


BLK = 4096


def _full(shape):
    return pl.BlockSpec(shape, lambda i: (0, 0))


def _t2_body(ufT, efT, both, uw1, ub1, uw2, ub2, ew1, eb1, ew2, eb2, aw, ab,
             ratingT, ipT, icT):
    f32 = jnp.float32
    h = jnp.maximum(
        lax.dot_general(uw1[...], ufT[...], _DN0, preferred_element_type=f32)
        + ub1[...], 0.0)
    ue = jnp.maximum(
        lax.dot_general(uw2[...], h, _DN0, preferred_element_type=f32)
        + ub2[...], 0.0)
    g = jnp.maximum(
        lax.dot_general(ew1[...], efT[...], _DN0, preferred_element_type=f32)
        + eb1[...], 0.0)
    ee = jnp.maximum(
        lax.dot_general(ew2[...], g, _DN0, preferred_element_type=f32)
        + eb2[...], 0.0)
    z = 0.5 * ue + 0.3 * ee
    bothT = jnp.swapaxes(both[...], 0, 1)
    ipt = bothT[:D]
    ict = bothT[D:]
    ipT[...] = ipt
    icT[...] = ict
    logits = lax.dot_general(
        aw[...], ipt + ict + z, _DN0, preferred_element_type=f32
    ) + ab[...]
    ratingT[...] = jax.nn.sigmoid(logits)


_t2 = pl.pallas_call(
    _t2_body,
    grid=(B // BLK,),
    in_specs=[
        pl.BlockSpec((UF, BLK), lambda i: (0, i)),
        pl.BlockSpec((EF, BLK), lambda i: (0, i)),
        pl.BlockSpec((BLK, 2 * D), lambda i: (i, 0)),
        _full((UF, 64)),
        _full((64, 1)),
        _full((64, D)),
        _full((D, 1)),
        _full((EF, 64)),
        _full((64, 1)),
        _full((64, D)),
        _full((D, 1)),
        _full((D, 1)),
        _full((1, 1)),
    ],
    out_specs=[
        pl.BlockSpec((1, BLK), lambda i: (0, i)),
        pl.BlockSpec((D, BLK), lambda i: (0, i)),
        pl.BlockSpec((D, BLK), lambda i: (0, i)),
    ],
    out_shape=[
        jax.ShapeDtypeStruct((1, B), jnp.float32),
        jax.ShapeDtypeStruct((D, B), jnp.float32),
        jax.ShapeDtypeStruct((D, B), jnp.float32),
    ],
)


def kernel(item_indices, user_features, exercise_features,
           item_personality, item_commonality,
           u_w1, u_b1, u_w2, u_b2,
           e_w1, e_b1, e_w2, e_b2,
           a_w, a_b):
    idx2 = item_indices.astype(jnp.int32).reshape(B // CHUNK, CHUNK)
    cat = _t1(item_personality.T, item_commonality.T)
    both = _sc_gather(idx2, cat)
    ratingT, ipT, icT = _t2(
        user_features.T, exercise_features.T, both,
        u_w1, u_b1.reshape(64, 1), u_w2, u_b2.reshape(D, 1),
        e_w1, e_b1.reshape(64, 1), e_w2, e_b2.reshape(D, 1),
        a_w, a_b.reshape(1, 1),
    )
    return (ratingT.T, ipT.T, icT.T)


# speedup vs baseline: 1.6054x; 1.0706x over previous
import functools

import jax
import jax.numpy as jnp
from jax import lax
from jax.experimental import pallas as pl
from jax.experimental.pallas import tpu as pltpu
from jax.experimental.pallas import tpu_sc as plsc

N = 100000
B = 16384
D = 64
UF = 13
EF = 13
NC = 2
NS = 16
NW = NC * NS
CHUNK = 128
CPW = B // (NW * CHUNK)
BPW = B // NW

_mesh = plsc.VectorSubcoreMesh(core_axis_name="c", subcore_axis_name="s")


@functools.partial(
    pl.kernel,
    mesh=_mesh,
    out_type=jax.ShapeDtypeStruct((B, 2 * D), jnp.float32),
    scratch_types=[
        pltpu.VMEM((CPW, CHUNK), jnp.int32),
        pltpu.VMEM((BPW, 2 * D), jnp.float32),
        pltpu.SemaphoreType.DMA,
    ],
    compiler_params=pltpu.CompilerParams(use_tc_tiling_on_sc=True),
)
def _sc_gather(idx_hbm, cat_hbm, out_hbm, idx_v, rows_v, sem):
    wid = lax.axis_index("s") * NC + lax.axis_index("c")
    base = wid * BPW
    pltpu.sync_copy(idx_hbm.at[pl.ds(wid * CPW, CPW)], idx_v)
    copies = []
    for j in range(CPW):
        copies.append(
            pltpu.async_copy(
                cat_hbm.at[idx_v.at[j]], rows_v.at[pl.ds(j * CHUNK, CHUNK)], sem
            )
        )
    for cp in copies:
        cp.wait()
    pltpu.sync_copy(rows_v, out_hbm.at[pl.ds(base, BPW)])


TBLK = 8192
_DN0 = (((0,), (0,)), ((), ()))


def _t1_body(persT, commT, cat):
    f32 = jnp.float32
    eye = jnp.eye(D, dtype=f32)
    cat[:, :D] = lax.dot_general(persT[...], eye, _DN0, preferred_element_type=f32)
    cat[:, D:] = lax.dot_general(commT[...], eye, _DN0, preferred_element_type=f32)


_t1 = pl.pallas_call(
    _t1_body,
    grid=(pl.cdiv(N, TBLK),),
    in_specs=[
        pl.BlockSpec((D, TBLK), lambda i: (0, i)),
        pl.BlockSpec((D, TBLK), lambda i: (0, i)),
    ],
    out_specs=pl.BlockSpec((TBLK, 2 * D), lambda i: (i, 0)),
    out_shape=jax.ShapeDtypeStruct((N, 2 * D), jnp.float32),
)


BLK = 8192


def _full(shape):
    return pl.BlockSpec(shape, lambda i: (0, 0))


def _t2_body(ufT, efT, both, uw1, ub1, uw2, ub2, ew1, eb1, ew2, eb2, aw, ab,
             ratingT, ipT, icT):
    f32 = jnp.float32
    h = jnp.maximum(
        lax.dot_general(uw1[...], ufT[...], _DN0, preferred_element_type=f32)
        + ub1[...], 0.0)
    ue = jnp.maximum(
        lax.dot_general(uw2[...], h, _DN0, preferred_element_type=f32)
        + ub2[...], 0.0)
    g = jnp.maximum(
        lax.dot_general(ew1[...], efT[...], _DN0, preferred_element_type=f32)
        + eb1[...], 0.0)
    ee = jnp.maximum(
        lax.dot_general(ew2[...], g, _DN0, preferred_element_type=f32)
        + eb2[...], 0.0)
    z = 0.5 * ue + 0.3 * ee
    bothT = jnp.swapaxes(both[...], 0, 1)
    ipt = bothT[:D]
    ict = bothT[D:]
    ipT[...] = ipt
    icT[...] = ict
    logits = lax.dot_general(
        aw[...], ipt + ict + z, _DN0, preferred_element_type=f32
    ) + ab[...]
    ratingT[...] = jax.nn.sigmoid(logits)


_t2 = pl.pallas_call(
    _t2_body,
    grid=(B // BLK,),
    in_specs=[
        pl.BlockSpec((UF, BLK), lambda i: (0, i)),
        pl.BlockSpec((EF, BLK), lambda i: (0, i)),
        pl.BlockSpec((BLK, 2 * D), lambda i: (i, 0)),
        _full((UF, 64)),
        _full((64, 1)),
        _full((64, D)),
        _full((D, 1)),
        _full((EF, 64)),
        _full((64, 1)),
        _full((64, D)),
        _full((D, 1)),
        _full((D, 1)),
        _full((1, 1)),
    ],
    out_specs=[
        pl.BlockSpec((1, BLK), lambda i: (0, i)),
        pl.BlockSpec((D, BLK), lambda i: (0, i)),
        pl.BlockSpec((D, BLK), lambda i: (0, i)),
    ],
    out_shape=[
        jax.ShapeDtypeStruct((1, B), jnp.float32),
        jax.ShapeDtypeStruct((D, B), jnp.float32),
        jax.ShapeDtypeStruct((D, B), jnp.float32),
    ],
)


def kernel(item_indices, user_features, exercise_features,
           item_personality, item_commonality,
           u_w1, u_b1, u_w2, u_b2,
           e_w1, e_b1, e_w2, e_b2,
           a_w, a_b):
    idx2 = item_indices.astype(jnp.int32).reshape(B // CHUNK, CHUNK)
    cat = _t1(item_personality.T, item_commonality.T)
    both = _sc_gather(idx2, cat)
    ratingT, ipT, icT = _t2(
        user_features.T, exercise_features.T, both,
        u_w1, u_b1.reshape(64, 1), u_w2, u_b2.reshape(D, 1),
        e_w1, e_b1.reshape(64, 1), e_w2, e_b2.reshape(D, 1),
        a_w, a_b.reshape(1, 1),
    )
    return (ratingT.T, ipT.T, icT.T)


# speedup vs baseline: 1.6505x; 1.0281x over previous
import functools

import jax
import jax.numpy as jnp
from jax import lax
from jax.experimental import pallas as pl
from jax.experimental.pallas import tpu as pltpu
from jax.experimental.pallas import tpu_sc as plsc

N = 100000
B = 16384
D = 64
UF = 13
EF = 13
NC = 2
NS = 16
NW = NC * NS
CHUNK = 128
CPW = B // (NW * CHUNK)
BPW = B // NW

_mesh = plsc.VectorSubcoreMesh(core_axis_name="c", subcore_axis_name="s")


@functools.partial(
    pl.kernel,
    mesh=_mesh,
    out_type=jax.ShapeDtypeStruct((B, 2 * D), jnp.float32),
    scratch_types=[
        pltpu.VMEM((CPW, CHUNK), jnp.int32),
        pltpu.VMEM((BPW, 2 * D), jnp.float32),
        pltpu.SemaphoreType.DMA,
    ],
    compiler_params=pltpu.CompilerParams(use_tc_tiling_on_sc=True),
)
def _sc_gather(idx_hbm, cat_hbm, out_hbm, idx_v, rows_v, sem):
    wid = lax.axis_index("s") * NC + lax.axis_index("c")
    base = wid * BPW
    pltpu.sync_copy(idx_hbm.at[pl.ds(wid * CPW, CPW)], idx_v)
    copies = []
    for j in range(CPW):
        copies.append(
            pltpu.async_copy(
                cat_hbm.at[idx_v.at[j]], rows_v.at[pl.ds(j * CHUNK, CHUNK)], sem
            )
        )
    for cp in copies:
        cp.wait()
    pltpu.sync_copy(rows_v, out_hbm.at[pl.ds(base, BPW)])


TBLK = 12800
_DN0 = (((0,), (0,)), ((), ()))


def _t1_body(persT, commT, cat):
    f32 = jnp.float32
    eye = jnp.eye(D, dtype=f32)
    cat[:, :D] = lax.dot_general(persT[...], eye, _DN0, preferred_element_type=f32)
    cat[:, D:] = lax.dot_general(commT[...], eye, _DN0, preferred_element_type=f32)


_t1 = pl.pallas_call(
    _t1_body,
    grid=(pl.cdiv(N, TBLK),),
    in_specs=[
        pl.BlockSpec((D, TBLK), lambda i: (0, i)),
        pl.BlockSpec((D, TBLK), lambda i: (0, i)),
    ],
    out_specs=pl.BlockSpec((TBLK, 2 * D), lambda i: (i, 0)),
    out_shape=jax.ShapeDtypeStruct((N, 2 * D), jnp.float32),
)


BLK = 8192


def _full(shape):
    return pl.BlockSpec(shape, lambda i: (0, 0))


def _t2_body(ufT, efT, both, uw1, ub1, uw2, ub2, ew1, eb1, ew2, eb2, aw, ab,
             ratingT, ipT, icT):
    f32 = jnp.float32
    h = jnp.maximum(
        lax.dot_general(uw1[...], ufT[...], _DN0, preferred_element_type=f32)
        + ub1[...], 0.0)
    ue = jnp.maximum(
        lax.dot_general(uw2[...], h, _DN0, preferred_element_type=f32)
        + ub2[...], 0.0)
    g = jnp.maximum(
        lax.dot_general(ew1[...], efT[...], _DN0, preferred_element_type=f32)
        + eb1[...], 0.0)
    ee = jnp.maximum(
        lax.dot_general(ew2[...], g, _DN0, preferred_element_type=f32)
        + eb2[...], 0.0)
    z = 0.5 * ue + 0.3 * ee
    bothT = jnp.swapaxes(both[...], 0, 1)
    ipt = bothT[:D]
    ict = bothT[D:]
    ipT[...] = ipt
    icT[...] = ict
    logits = lax.dot_general(
        aw[...], ipt + ict + z, _DN0, preferred_element_type=f32
    ) + ab[...]
    ratingT[...] = jax.nn.sigmoid(logits)


_t2 = pl.pallas_call(
    _t2_body,
    grid=(B // BLK,),
    in_specs=[
        pl.BlockSpec((UF, BLK), lambda i: (0, i)),
        pl.BlockSpec((EF, BLK), lambda i: (0, i)),
        pl.BlockSpec((BLK, 2 * D), lambda i: (i, 0)),
        _full((UF, 64)),
        _full((64, 1)),
        _full((64, D)),
        _full((D, 1)),
        _full((EF, 64)),
        _full((64, 1)),
        _full((64, D)),
        _full((D, 1)),
        _full((D, 1)),
        _full((1, 1)),
    ],
    out_specs=[
        pl.BlockSpec((1, BLK), lambda i: (0, i)),
        pl.BlockSpec((D, BLK), lambda i: (0, i)),
        pl.BlockSpec((D, BLK), lambda i: (0, i)),
    ],
    out_shape=[
        jax.ShapeDtypeStruct((1, B), jnp.float32),
        jax.ShapeDtypeStruct((D, B), jnp.float32),
        jax.ShapeDtypeStruct((D, B), jnp.float32),
    ],
)


def kernel(item_indices, user_features, exercise_features,
           item_personality, item_commonality,
           u_w1, u_b1, u_w2, u_b2,
           e_w1, e_b1, e_w2, e_b2,
           a_w, a_b):
    idx2 = item_indices.astype(jnp.int32).reshape(B // CHUNK, CHUNK)
    cat = _t1(item_personality.T, item_commonality.T)
    both = _sc_gather(idx2, cat)
    ratingT, ipT, icT = _t2(
        user_features.T, exercise_features.T, both,
        u_w1, u_b1.reshape(64, 1), u_w2, u_b2.reshape(D, 1),
        e_w1, e_b1.reshape(64, 1), e_w2, e_b2.reshape(D, 1),
        a_w, a_b.reshape(1, 1),
    )
    return (ratingT.T, ipT.T, icT.T)
